# baseline (speedup 1.0000x reference)
import functools

import jax
import jax.numpy as jnp
from jax import lax
from jax.experimental import pallas as pl
from jax.experimental.pallas import tpu as pltpu
from jax.experimental.pallas import tpu_sc as plsc

N_NODES = 10000
N_EDGES = 320000
D_IN = 128
D_HID = 128
D_OUT = 64
N_GRAPHS = 64

NC = 2
NS = 16
L = 16
NW = NC * NS
N_PAD = 10240
NODES_PER_W = N_PAD // NW
SEGE = 4000
NSEG = N_EDGES // SEGE
NGRP = SEGE // L
CAP = 4096
NCHUNK = D_HID // L
BLKE = 128


def _proj_body(x_ref, wl_ref, wr_ref, xl_ref, xr_ref):
    x = x_ref[...]
    xl_ref[...] = jnp.dot(x, wl_ref[...], preferred_element_type=jnp.float32)
    xr_ref[...] = jnp.dot(x, wr_ref[...], preferred_element_type=jnp.float32)


_proj = pl.pallas_call(
    _proj_body,
    out_shape=(
        jax.ShapeDtypeStruct((N_NODES, D_HID), jnp.float32),
        jax.ShapeDtypeStruct((N_NODES, D_HID), jnp.float32),
    ),
)


def _edge_body(xl_hbm, xr_hbm, src_hbm, dst_hbm, att_hbm,
               num_out, den_out,
               segs, segd, csrc, cdst, xlr, xrr,
               accb, wb, attv, num_l, den_l, sem1, sem2):
    cid = lax.axis_index("c")
    sid = lax.axis_index("s")
    wid = sid * NC + cid
    lo = wid * NODES_PER_W
    zf = jnp.zeros((L,), jnp.float32)
    zi = jnp.zeros((L,), jnp.int32)
    iota = lax.iota(jnp.int32, L)

    def znum_body(i, c0):
        num_l[pl.ds(i * L, L)] = zf
        return c0

    lax.fori_loop(0, NODES_PER_W * NCHUNK, znum_body, 0)

    def zden_body(i, c0):
        den_l[pl.ds(i * L, L)] = zf
        return c0

    lax.fori_loop(0, NODES_PER_W // L, zden_body, 0)

    def zidx_body(i, c0):
        csrc[pl.ds(i * L, L)] = zi
        cdst[pl.ds(i * L, L)] = zi
        return c0

    lax.fori_loop(0, CAP // L, zidx_body, 0)
    pltpu.sync_copy(att_hbm, attv)

    att_chunks = [attv[pl.ds(c * L, L)] for c in range(NCHUNK)]

    def seg_body(s, carry):
        e0 = s * SEGE
        pltpu.sync_copy(src_hbm.at[pl.ds(e0, SEGE)], segs)
        pltpu.sync_copy(dst_hbm.at[pl.ds(e0, SEGE)], segd)

        def scan_body(g, cnt):
            d16 = segd[pl.ds(g * L, L)]
            s16 = segs[pl.ds(g * L, L)]
            dl = d16 - lo
            mask = (dl >= 0) & (dl < NODES_PER_W)
            ones = jnp.where(mask, 1, 0)
            csum = plsc.cumsum(ones)
            pos = (zi + cnt) + csum - ones
            plsc.store_scatter(csrc, [pos], s16, mask=mask)
            plsc.store_scatter(cdst, [pos], d16, mask=mask)
            return cnt + lax.reduce_max(csum, (0,))

        cnt = lax.fori_loop(0, NGRP, scan_body, 0)

        def blk_body(b, c0):
            d1 = pltpu.async_copy(xl_hbm.at[csrc.at[pl.ds(b * BLKE, BLKE)]],
                                  xlr, sem1)
            d2 = pltpu.async_copy(xr_hbm.at[cdst.at[pl.ds(b * BLKE, BLKE)]],
                                  xrr, sem2)
            d1.wait()
            d2.wait()

            def score_body(i, c1):
                acc = zf
                for c in range(NCHUNK):
                    sv = xlr[i, pl.ds(c * L, L)] + xrr[i, pl.ds(c * L, L)]
                    acc = acc + att_chunks[c] * jnp.maximum(sv, 0.2 * sv)
                accb[i, :] = acc
                return c1

            lax.fori_loop(0, BLKE, score_body, 0)

            for g in range(BLKE // L):
                rows = g * L + iota
                e16 = zf
                for c in range(L):
                    e16 = e16 + plsc.load_gather(accb, [rows, zi + c])
                u = e16 * (1.0 / 64.0)
                p = 1.0 + u * (1.0 + u * (0.5 + u * (
                    (1.0 / 6.0) + u * ((1.0 / 24.0) + u * (1.0 / 120.0)))))
                for _sq in range(6):
                    p = p * p
                valid = (b * BLKE + rows) < cnt
                wb[pl.ds(g * L, L)] = jnp.where(valid, p, 0.0)

            def acc_body(i, c1):
                wbc = plsc.load_gather(wb, [zi + i])
                dn16 = jnp.clip(plsc.load_gather(cdst, [zi + b * BLKE + i])
                                - lo, 0, NODES_PER_W - 1)
                plsc.addupdate_scatter(den_l, [dn16], wbc, mask=iota == 0)
                dbase = dn16 * D_HID
                for c in range(NCHUNK):
                    chunk = wbc * xlr[i, pl.ds(c * L, L)]
                    plsc.addupdate_scatter(num_l, [dbase + c * L + iota],
                                           chunk)
                return c1

            lax.fori_loop(0, BLKE, acc_body, 0)
            return c0

        nblk = (cnt + (BLKE - 1)) // BLKE
        lax.fori_loop(0, nblk, blk_body, 0)
        return carry

    lax.fori_loop(0, NSEG, seg_body, 0)

    pltpu.sync_copy(num_l, num_out.at[pl.ds(lo * D_HID, NODES_PER_W * D_HID)])
    pltpu.sync_copy(den_l, den_out.at[pl.ds(lo, NODES_PER_W)])


_edge_kernel = functools.partial(
    pl.kernel,
    out_type=(
        jax.ShapeDtypeStruct((N_PAD * D_HID,), jnp.float32),
        jax.ShapeDtypeStruct((N_PAD,), jnp.float32),
    ),
    mesh=plsc.VectorSubcoreMesh(core_axis_name="c", subcore_axis_name="s"),
    compiler_params=pltpu.CompilerParams(needs_layout_passes=False),
    scratch_types=[
        pltpu.VMEM((SEGE,), jnp.int32),
        pltpu.VMEM((SEGE,), jnp.int32),
        pltpu.VMEM((CAP,), jnp.int32),
        pltpu.VMEM((CAP,), jnp.int32),
        pltpu.VMEM((BLKE, D_HID), jnp.float32),
        pltpu.VMEM((BLKE, D_HID), jnp.float32),
        pltpu.VMEM((BLKE, L), jnp.float32),
        pltpu.VMEM((BLKE,), jnp.float32),
        pltpu.VMEM((D_HID,), jnp.float32),
        pltpu.VMEM((NODES_PER_W * D_HID,), jnp.float32),
        pltpu.VMEM((NODES_PER_W,), jnp.float32),
        pltpu.SemaphoreType.DMA,
        pltpu.SemaphoreType.DMA,
    ],
)(_edge_body)


def _final_body(num_ref, den_ref, batch_ref, bconv_ref, fcw_ref, fcb_ref,
                out_ref):
    num = num_ref[:N_NODES]
    den = den_ref[:N_NODES]
    out = num / (den + 1e-16) + bconv_ref[...]
    gids = lax.broadcasted_iota(jnp.int32, (N_GRAPHS, N_NODES), 0)
    m = (batch_ref[...] == gids).astype(jnp.float32)
    sums = jnp.dot(m, out, preferred_element_type=jnp.float32)
    counts = jnp.dot(m, jnp.ones((N_NODES, 1), jnp.float32),
                     preferred_element_type=jnp.float32)
    pooled = sums / jnp.maximum(counts, 1.0)
    h = jnp.where(pooled > 0, pooled, 0.01 * pooled)
    out_ref[...] = (jnp.dot(h, fcw_ref[...], preferred_element_type=jnp.float32)
                    + fcb_ref[...])


_final = pl.pallas_call(
    _final_body,
    out_shape=jax.ShapeDtypeStruct((N_GRAPHS, D_OUT), jnp.float32),
)


def kernel(x, edge_index, batch, add_features, W_l, W_r, att, b_conv, fc_W,
           fc_b):
    xl, xr = _proj(x, W_l, W_r)
    src = edge_index[0].astype(jnp.int32)
    dst = edge_index[1].astype(jnp.int32)
    numf, denf = _edge_kernel(xl, xr, src, dst, att)
    num2 = numf.reshape(N_PAD, D_HID)
    den2 = denf.reshape(N_PAD, 1)
    batch_row = batch.astype(jnp.int32).reshape(1, N_NODES)
    return _final(num2, den2, batch_row, b_conv.reshape(1, D_HID), fc_W,
                  fc_b.reshape(1, D_OUT))


# speedup vs baseline: 2.3691x; 2.3691x over previous
import functools

import jax
import jax.numpy as jnp
from jax import lax
from jax.experimental import pallas as pl
from jax.experimental.pallas import tpu as pltpu
from jax.experimental.pallas import tpu_sc as plsc

N_NODES = 10000
N_EDGES = 320000
D_IN = 128
D_HID = 128
D_OUT = 64
N_GRAPHS = 64

NC = 2
NS = 16
L = 16
NW = NC * NS
N_PAD = 10240
NODES_PER_W = N_PAD // NW
SEGE = 3200
NSEG = N_EDGES // SEGE
NGRP = SEGE // L
CAP = 3200
NCHUNK = D_HID // L
BLKE = 128


def _proj_body(x_ref, wl_ref, wr_ref, xl_ref, xr_ref):
    x = x_ref[...]
    xl_ref[...] = jnp.dot(x, wl_ref[...], preferred_element_type=jnp.float32)
    xr_ref[...] = jnp.dot(x, wr_ref[...], preferred_element_type=jnp.float32)


_proj = pl.pallas_call(
    _proj_body,
    out_shape=(
        jax.ShapeDtypeStruct((N_NODES, D_HID), jnp.float32),
        jax.ShapeDtypeStruct((N_NODES, D_HID), jnp.float32),
    ),
)


def _edge_body(xl_hbm, xr_hbm, src_hbm, dst_hbm, att_hbm,
               num_out, den_out,
               segs, segd, csrc, cdst, sidx, didx, xlr, xr_l,
               accb, wb, attv, num_l, den_l, sem1, sem2):
    cid = lax.axis_index("c")
    sid = lax.axis_index("s")
    wid = sid * NC + cid
    lo = wid * NODES_PER_W
    zf = jnp.zeros((L,), jnp.float32)
    zi = jnp.zeros((L,), jnp.int32)
    iota = lax.iota(jnp.int32, L)

    def znum_body(i, c0):
        num_l[pl.ds(i * L, L)] = zf
        return c0

    lax.fori_loop(0, NODES_PER_W * NCHUNK, znum_body, 0)

    def zden_body(i, c0):
        den_l[pl.ds(i * L, L)] = zf
        return c0

    lax.fori_loop(0, NODES_PER_W // L, zden_body, 0)

    def zidx_body(i, c0):
        csrc[pl.ds(i * L, L)] = zi
        cdst[pl.ds(i * L, L)] = zi
        return c0

    lax.fori_loop(0, CAP // L, zidx_body, 0)
    pltpu.sync_copy(att_hbm, attv)
    pltpu.sync_copy(xr_hbm.at[pl.ds(lo, NODES_PER_W)], xr_l)

    att_chunks = [attv[pl.ds(c * L, L)] for c in range(NCHUNK)]

    def seg_body(s, carry):
        e0 = s * SEGE
        pltpu.sync_copy(src_hbm.at[pl.ds(e0, SEGE)], segs)
        pltpu.sync_copy(dst_hbm.at[pl.ds(e0, SEGE)], segd)

        def scan_body(g, cnt):
            d16 = segd[pl.ds(g * L, L)]
            s16 = segs[pl.ds(g * L, L)]
            dl = d16 - lo
            mask = (dl >= 0) & (dl < NODES_PER_W)
            ones = jnp.where(mask, 1, 0)
            csum = plsc.cumsum(ones)
            pos = (zi + cnt) + csum - ones
            plsc.store_scatter(csrc, [pos], s16, mask=mask)
            plsc.store_scatter(cdst, [pos], d16, mask=mask)
            return cnt + lax.reduce_max(csum, (0,))

        cnt = lax.fori_loop(0, NGRP, scan_body, 0)

        def blk_body(b, c0):
            for k in range(BLKE // L):
                sidx[pl.ds(k * L, L)] = csrc[pl.ds(b * BLKE + k * L, L)]
                didx[pl.ds(k * L, L)] = jnp.clip(
                    cdst[pl.ds(b * BLKE + k * L, L)] - lo, 0,
                    NODES_PER_W - 1)
            d1 = pltpu.async_copy(xl_hbm.at[sidx], xlr, sem1)
            d1.wait()

            def score_body(i, c1):
                dn16 = plsc.load_gather(didx, [zi + i])
                acc = zf
                for c in range(NCHUNK):
                    rv = plsc.load_gather(xr_l, [dn16, c * L + iota])
                    sv = xlr[i, pl.ds(c * L, L)] + rv
                    acc = acc + att_chunks[c] * jnp.maximum(sv, 0.2 * sv)
                accb[i, :] = acc
                return c1

            lax.fori_loop(0, BLKE, score_body, 0)

            for g in range(BLKE // L):
                rows = g * L + iota
                e16 = zf
                for c in range(L):
                    e16 = e16 + plsc.load_gather(accb, [rows, zi + c])
                u = e16 * (1.0 / 64.0)
                p = 1.0 + u * (1.0 + u * (0.5 + u * (
                    (1.0 / 6.0) + u * ((1.0 / 24.0) + u * (1.0 / 120.0)))))
                for _sq in range(6):
                    p = p * p
                valid = (b * BLKE + rows) < cnt
                wb[pl.ds(g * L, L)] = jnp.where(valid, p, 0.0)

            def acc_body(i, c1):
                wbc = plsc.load_gather(wb, [zi + i])
                dn16 = plsc.load_gather(didx, [zi + i])
                plsc.addupdate_scatter(den_l, [dn16], wbc, mask=iota == 0)
                dbase = dn16 * D_HID
                for c in range(NCHUNK):
                    chunk = wbc * xlr[i, pl.ds(c * L, L)]
                    plsc.addupdate_scatter(num_l, [dbase + c * L + iota],
                                           chunk)
                return c1

            lax.fori_loop(0, BLKE, acc_body, 0)
            return c0

        nblk = (cnt + (BLKE - 1)) // BLKE
        lax.fori_loop(0, nblk, blk_body, 0)
        return carry

    lax.fori_loop(0, NSEG, seg_body, 0)

    pltpu.sync_copy(num_l, num_out.at[pl.ds(lo * D_HID, NODES_PER_W * D_HID)])
    pltpu.sync_copy(den_l, den_out.at[pl.ds(lo, NODES_PER_W)])


_edge_kernel = functools.partial(
    pl.kernel,
    out_type=(
        jax.ShapeDtypeStruct((N_PAD * D_HID,), jnp.float32),
        jax.ShapeDtypeStruct((N_PAD,), jnp.float32),
    ),
    mesh=plsc.VectorSubcoreMesh(core_axis_name="c", subcore_axis_name="s"),
    compiler_params=pltpu.CompilerParams(needs_layout_passes=False),
    scratch_types=[
        pltpu.VMEM((SEGE,), jnp.int32),
        pltpu.VMEM((SEGE,), jnp.int32),
        pltpu.VMEM((CAP,), jnp.int32),
        pltpu.VMEM((CAP,), jnp.int32),
        pltpu.VMEM((BLKE,), jnp.int32),
        pltpu.VMEM((BLKE,), jnp.int32),
        pltpu.VMEM((BLKE, D_HID), jnp.float32),
        pltpu.VMEM((NODES_PER_W, D_HID), jnp.float32),
        pltpu.VMEM((BLKE, L), jnp.float32),
        pltpu.VMEM((BLKE,), jnp.float32),
        pltpu.VMEM((D_HID,), jnp.float32),
        pltpu.VMEM((NODES_PER_W * D_HID,), jnp.float32),
        pltpu.VMEM((NODES_PER_W,), jnp.float32),
        pltpu.SemaphoreType.DMA,
        pltpu.SemaphoreType.DMA,
    ],
)(_edge_body)


def _final_body(num_ref, den_ref, batch_ref, bconv_ref, fcw_ref, fcb_ref,
                out_ref):
    num = num_ref[:N_NODES]
    den = den_ref[:N_NODES]
    out = num / (den + 1e-16) + bconv_ref[...]
    gids = lax.broadcasted_iota(jnp.int32, (N_GRAPHS, N_NODES), 0)
    m = (batch_ref[...] == gids).astype(jnp.float32)
    sums = jnp.dot(m, out, preferred_element_type=jnp.float32)
    counts = jnp.dot(m, jnp.ones((N_NODES, 1), jnp.float32),
                     preferred_element_type=jnp.float32)
    pooled = sums / jnp.maximum(counts, 1.0)
    h = jnp.where(pooled > 0, pooled, 0.01 * pooled)
    out_ref[...] = (jnp.dot(h, fcw_ref[...], preferred_element_type=jnp.float32)
                    + fcb_ref[...])


_final = pl.pallas_call(
    _final_body,
    out_shape=jax.ShapeDtypeStruct((N_GRAPHS, D_OUT), jnp.float32),
)


def kernel(x, edge_index, batch, add_features, W_l, W_r, att, b_conv, fc_W,
           fc_b):
    xl, xr = _proj(x, W_l, W_r)
    xr_pad = jnp.concatenate(
        [xr, jnp.zeros((N_PAD - N_NODES, D_HID), jnp.float32)])
    src = edge_index[0].astype(jnp.int32)
    dst = edge_index[1].astype(jnp.int32)
    numf, denf = _edge_kernel(xl, xr_pad, src, dst, att)
    num2 = numf.reshape(N_PAD, D_HID)
    den2 = denf.reshape(N_PAD, 1)
    batch_row = batch.astype(jnp.int32).reshape(1, N_NODES)
    return _final(num2, den2, batch_row, b_conv.reshape(1, D_HID), fc_W,
                  fc_b.reshape(1, D_OUT))
